# baseline (device time: 114237 ns/iter reference)
import jax
import jax.numpy as jnp
from jax import lax
from jax.experimental import pallas as pl
from jax.experimental.pallas import tpu as pltpu

N_DEV = 4
CHUNK = 16

_CompilerParams = getattr(pltpu, "CompilerParams", None) or getattr(
    pltpu, "TPUCompilerParams"
)


def _neighbor_barrier(me):
    barrier = pltpu.get_barrier_semaphore()
    for d in range(1, N_DEV):
        pl.semaphore_signal(
            barrier,
            inc=1,
            device_id=((me + d) % N_DEV,),
            device_id_type=pl.DeviceIdType.MESH,
        )
    pl.semaphore_wait(barrier, N_DEV - 1)


def _counts_body(cnt_ref, out_ref, local_sem, send_sems, recv_sems):
    me = lax.axis_index("i")
    _neighbor_barrier(me)

    own = pltpu.make_async_copy(cnt_ref, out_ref.at[pl.ds(me, 1)], local_sem)
    own.start()

    rdmas = []
    for d in range(1, N_DEV):
        p = (me + d) % N_DEV
        r = pltpu.make_async_remote_copy(
            src_ref=cnt_ref,
            dst_ref=out_ref.at[pl.ds(me, 1)],
            send_sem=send_sems.at[d - 1],
            recv_sem=recv_sems.at[d - 1],
            device_id=(p,),
            device_id_type=pl.DeviceIdType.MESH,
        )
        r.start()
        rdmas.append(r)

    own.wait()
    for r in rdmas:
        r.wait()


def _gather_counts(counts_row):
    return pl.pallas_call(
        _counts_body,
        out_shape=jax.ShapeDtypeStruct((N_DEV, 8, 128), jnp.int32),
        in_specs=[pl.BlockSpec(memory_space=pltpu.VMEM)],
        out_specs=pl.BlockSpec(memory_space=pltpu.VMEM),
        scratch_shapes=[
            pltpu.SemaphoreType.DMA,
            pltpu.SemaphoreType.DMA((N_DEV - 1,)),
            pltpu.SemaphoreType.DMA((N_DEV - 1,)),
        ],
        compiler_params=_CompilerParams(collective_id=0),
    )(counts_row)


def _a2a_body(
    x_ref,
    cm_ref,
    out_ref,
    send_big,
    send_small,
    recv_big,
    recv_small,
    local_big,
    local_small,
):
    me = lax.axis_index("i")
    _neighbor_barrier(me)

    C = CHUNK
    zero = jnp.int32(0)

    def lstart(p):
        acc = zero
        for d in range(N_DEV):
            acc = acc + jnp.where(jnp.int32(d) < p, cm_ref[me, d], zero)
        return acc

    def rbase(p):
        acc = zero
        for s in range(N_DEV):
            acc = acc + jnp.where(jnp.int32(s) < me, cm_ref[s, p], zero)
        return acc

    nb_send = zero
    ns_send = zero
    for d in range(1, N_DEV):
        p = (me + d) % N_DEV
        c = cm_ref[me, p]
        s0 = lstart(p)
        t0 = rbase(p)
        nb = c // C
        rem = c - nb * C

        def send_chunk(j, _, p=p, s0=s0, t0=t0):
            pltpu.make_async_remote_copy(
                src_ref=x_ref.at[pl.ds(s0 + j * C, C)],
                dst_ref=out_ref.at[pl.ds(t0 + j * C, C)],
                send_sem=send_big,
                recv_sem=recv_big,
                device_id=(p,),
                device_id_type=pl.DeviceIdType.MESH,
            ).start()
            return 0

        lax.fori_loop(0, nb, send_chunk, 0)

        def send_row(k, _, p=p, s0=s0, t0=t0, base=nb * C):
            pltpu.make_async_remote_copy(
                src_ref=x_ref.at[pl.ds(s0 + base + k, 1)],
                dst_ref=out_ref.at[pl.ds(t0 + base + k, 1)],
                send_sem=send_small,
                recv_sem=recv_small,
                device_id=(p,),
                device_id_type=pl.DeviceIdType.MESH,
            ).start()
            return 0

        lax.fori_loop(0, rem, send_row, 0)
        nb_send = nb_send + nb
        ns_send = ns_send + rem

    c_self = cm_ref[me, me]
    s0_self = lstart(me)
    t0_self = rbase(me)
    nb_self = c_self // C
    rem_self = c_self - nb_self * C
    base_self = nb_self * C

    def loc_chunk(j, _):
        pltpu.make_async_copy(
            x_ref.at[pl.ds(s0_self + j * C, C)],
            out_ref.at[pl.ds(t0_self + j * C, C)],
            local_big,
        ).start()
        return 0

    lax.fori_loop(0, nb_self, loc_chunk, 0)

    def loc_row(k, _):
        pltpu.make_async_copy(
            x_ref.at[pl.ds(s0_self + base_self + k, 1)],
            out_ref.at[pl.ds(t0_self + base_self + k, 1)],
            local_small,
        ).start()
        return 0

    lax.fori_loop(0, rem_self, loc_row, 0)

    big_dummy = pltpu.make_async_remote_copy(
        src_ref=x_ref.at[pl.ds(0, C)],
        dst_ref=out_ref.at[pl.ds(0, C)],
        send_sem=send_big,
        recv_sem=recv_big,
        device_id=(me,),
        device_id_type=pl.DeviceIdType.MESH,
    )
    small_dummy = pltpu.make_async_remote_copy(
        src_ref=x_ref.at[pl.ds(0, 1)],
        dst_ref=out_ref.at[pl.ds(0, 1)],
        send_sem=send_small,
        recv_sem=recv_small,
        device_id=(me,),
        device_id_type=pl.DeviceIdType.MESH,
    )

    def wait_send_big(i, _):
        big_dummy.wait_send()
        return 0

    def wait_send_small(i, _):
        small_dummy.wait_send()
        return 0

    def wait_recv_big(i, _):
        big_dummy.wait_recv()
        return 0

    def wait_recv_small(i, _):
        small_dummy.wait_recv()
        return 0

    nbig_in = zero
    nsmall_in = zero
    for s in range(N_DEV):
        c_in = cm_ref[s, me]
        is_remote = jnp.int32(s) != me
        nbig_in = nbig_in + jnp.where(is_remote, c_in // C, zero)
        nsmall_in = nsmall_in + jnp.where(is_remote, c_in % C, zero)

    lax.fori_loop(0, nb_send, wait_send_big, 0)
    lax.fori_loop(0, ns_send, wait_send_small, 0)
    lax.fori_loop(0, nbig_in, wait_recv_big, 0)
    lax.fori_loop(0, nsmall_in, wait_recv_small, 0)

    def wait_loc_big(i, _):
        pltpu.make_async_copy(
            x_ref.at[pl.ds(0, C)], out_ref.at[pl.ds(0, C)], local_big
        ).wait()
        return 0

    def wait_loc_small(i, _):
        pltpu.make_async_copy(
            x_ref.at[pl.ds(0, 1)], out_ref.at[pl.ds(0, 1)], local_small
        ).wait()
        return 0

    lax.fori_loop(0, nb_self, wait_loc_big, 0)
    lax.fori_loop(0, rem_self, wait_loc_small, 0)


def _a2a(x_sorted, cm):
    return pl.pallas_call(
        _a2a_body,
        out_shape=jax.ShapeDtypeStruct(x_sorted.shape, x_sorted.dtype),
        in_specs=[
            pl.BlockSpec(memory_space=pltpu.VMEM),
            pl.BlockSpec(memory_space=pltpu.SMEM),
        ],
        out_specs=pl.BlockSpec(memory_space=pltpu.VMEM),
        scratch_shapes=[
            pltpu.SemaphoreType.DMA,
            pltpu.SemaphoreType.DMA,
            pltpu.SemaphoreType.DMA,
            pltpu.SemaphoreType.DMA,
            pltpu.SemaphoreType.DMA,
            pltpu.SemaphoreType.DMA,
        ],
        compiler_params=_CompilerParams(collective_id=1),
    )(x_sorted, cm)


def kernel(x, dest):
    me = lax.axis_index("i")
    dest = dest.astype(jnp.int32)

    m, n = x.shape
    oh = (
        dest[:, None] == jnp.arange(N_DEV, dtype=jnp.int32)[None, :]
    ).astype(jnp.int32)
    csum = jnp.cumsum(oh, axis=0)
    counts = csum[-1]
    occ = jnp.sum((csum - oh) * oh, axis=1)
    group_start = jnp.cumsum(counts) - counts
    scatter_idx = occ + group_start[dest]
    perm = (
        jnp.zeros((m,), jnp.int32)
        .at[scatter_idx]
        .set(jnp.arange(m, dtype=jnp.int32))
    )
    x_sorted = x.astype(jnp.bfloat16)[perm].reshape(m, 8, n // 8)
    counts_row = jnp.zeros((1, 8, 128), jnp.int32).at[0, 0, :N_DEV].set(counts)
    cm = _gather_counts(counts_row)[:, 0, :N_DEV]

    out = _a2a(x_sorted, cm)
    return out.reshape(m, n)


# device time: 95226 ns/iter; 1.1996x vs baseline; 1.1996x over previous
import jax
import jax.numpy as jnp
from jax import lax
from jax.experimental import pallas as pl
from jax.experimental.pallas import tpu as pltpu

N_DEV = 4
CHUNK = 16

_CompilerParams = getattr(pltpu, "CompilerParams", None) or getattr(
    pltpu, "TPUCompilerParams"
)


def _neighbor_barrier(me):
    barrier = pltpu.get_barrier_semaphore()
    for d in range(1, N_DEV):
        pl.semaphore_signal(
            barrier,
            inc=1,
            device_id=((me + d) % N_DEV,),
            device_id_type=pl.DeviceIdType.MESH,
        )
    pl.semaphore_wait(barrier, N_DEV - 1)


def _counts_body(cnt_ref, out_ref, local_sem, send_sems, recv_sems):
    me = lax.axis_index("i")
    _neighbor_barrier(me)

    own = pltpu.make_async_copy(cnt_ref, out_ref.at[pl.ds(me, 1)], local_sem)
    own.start()

    rdmas = []
    for d in range(1, N_DEV):
        p = (me + d) % N_DEV
        r = pltpu.make_async_remote_copy(
            src_ref=cnt_ref,
            dst_ref=out_ref.at[pl.ds(me, 1)],
            send_sem=send_sems.at[d - 1],
            recv_sem=recv_sems.at[d - 1],
            device_id=(p,),
            device_id_type=pl.DeviceIdType.MESH,
        )
        r.start()
        rdmas.append(r)

    own.wait()
    for r in rdmas:
        r.wait()


def _gather_counts(counts_row):
    return pl.pallas_call(
        _counts_body,
        out_shape=jax.ShapeDtypeStruct((N_DEV, 8, 128), jnp.int32),
        in_specs=[pl.BlockSpec(memory_space=pltpu.VMEM)],
        out_specs=pl.BlockSpec(memory_space=pltpu.VMEM),
        scratch_shapes=[
            pltpu.SemaphoreType.DMA,
            pltpu.SemaphoreType.DMA((N_DEV - 1,)),
            pltpu.SemaphoreType.DMA((N_DEV - 1,)),
        ],
        compiler_params=_CompilerParams(collective_id=0),
    )(counts_row)


def _a2a_body(
    x_ref,
    cm_ref,
    out_ref,
    send_big,
    send_small,
    recv_big,
    recv_small,
    local_big,
    local_small,
):
    me = lax.axis_index("i")
    _neighbor_barrier(me)

    C = CHUNK
    zero = jnp.int32(0)

    def lstart(p):
        acc = zero
        for d in range(N_DEV):
            acc = acc + jnp.where(jnp.int32(d) < p, cm_ref[me, d], zero)
        return acc

    def rbase(p):
        acc = zero
        for s in range(N_DEV):
            acc = acc + jnp.where(jnp.int32(s) < me, cm_ref[s, p], zero)
        return acc

    nb_send = zero
    ns_send = zero
    for d in range(1, N_DEV):
        p = (me + d) % N_DEV
        c = cm_ref[me, p]
        s0 = lstart(p)
        t0 = rbase(p)
        nb = c // C
        rem = c - nb * C

        def send_chunk(j, _, p=p, s0=s0, t0=t0):
            pltpu.make_async_remote_copy(
                src_ref=x_ref.at[pl.ds(s0 + j * C, C)],
                dst_ref=out_ref.at[pl.ds(t0 + j * C, C)],
                send_sem=send_big,
                recv_sem=recv_big,
                device_id=(p,),
                device_id_type=pl.DeviceIdType.MESH,
            ).start()
            return 0

        lax.fori_loop(0, nb, send_chunk, 0)

        def send_row(k, _, p=p, s0=s0, t0=t0, base=nb * C):
            pltpu.make_async_remote_copy(
                src_ref=x_ref.at[pl.ds(s0 + base + k, 1)],
                dst_ref=out_ref.at[pl.ds(t0 + base + k, 1)],
                send_sem=send_small,
                recv_sem=recv_small,
                device_id=(p,),
                device_id_type=pl.DeviceIdType.MESH,
            ).start()
            return 0

        lax.fori_loop(0, rem, send_row, 0)
        nb_send = nb_send + nb
        ns_send = ns_send + rem

    c_self = cm_ref[me, me]
    s0_self = lstart(me)
    t0_self = rbase(me)
    nb_self = c_self // C
    rem_self = c_self - nb_self * C
    base_self = nb_self * C

    def loc_chunk(j, _):
        pltpu.make_async_copy(
            x_ref.at[pl.ds(s0_self + j * C, C)],
            out_ref.at[pl.ds(t0_self + j * C, C)],
            local_big,
        ).start()
        return 0

    lax.fori_loop(0, nb_self, loc_chunk, 0)

    def loc_row(k, _):
        pltpu.make_async_copy(
            x_ref.at[pl.ds(s0_self + base_self + k, 1)],
            out_ref.at[pl.ds(t0_self + base_self + k, 1)],
            local_small,
        ).start()
        return 0

    lax.fori_loop(0, rem_self, loc_row, 0)

    big_dummy = pltpu.make_async_remote_copy(
        src_ref=x_ref.at[pl.ds(0, C)],
        dst_ref=out_ref.at[pl.ds(0, C)],
        send_sem=send_big,
        recv_sem=recv_big,
        device_id=(me,),
        device_id_type=pl.DeviceIdType.MESH,
    )
    small_dummy = pltpu.make_async_remote_copy(
        src_ref=x_ref.at[pl.ds(0, 1)],
        dst_ref=out_ref.at[pl.ds(0, 1)],
        send_sem=send_small,
        recv_sem=recv_small,
        device_id=(me,),
        device_id_type=pl.DeviceIdType.MESH,
    )

    def wait_send_big(i, _):
        big_dummy.wait_send()
        return 0

    def wait_send_small(i, _):
        small_dummy.wait_send()
        return 0

    def wait_recv_big(i, _):
        big_dummy.wait_recv()
        return 0

    def wait_recv_small(i, _):
        small_dummy.wait_recv()
        return 0

    nbig_in = zero
    nsmall_in = zero
    for s in range(N_DEV):
        c_in = cm_ref[s, me]
        is_remote = jnp.int32(s) != me
        nbig_in = nbig_in + jnp.where(is_remote, c_in // C, zero)
        nsmall_in = nsmall_in + jnp.where(is_remote, c_in % C, zero)

    lax.fori_loop(0, nb_send, wait_send_big, 0)
    lax.fori_loop(0, ns_send, wait_send_small, 0)
    lax.fori_loop(0, nbig_in, wait_recv_big, 0)
    lax.fori_loop(0, nsmall_in, wait_recv_small, 0)

    def wait_loc_big(i, _):
        pltpu.make_async_copy(
            x_ref.at[pl.ds(0, C)], out_ref.at[pl.ds(0, C)], local_big
        ).wait()
        return 0

    def wait_loc_small(i, _):
        pltpu.make_async_copy(
            x_ref.at[pl.ds(0, 1)], out_ref.at[pl.ds(0, 1)], local_small
        ).wait()
        return 0

    lax.fori_loop(0, nb_self, wait_loc_big, 0)
    lax.fori_loop(0, rem_self, wait_loc_small, 0)


def _a2a(x_sorted, cm):
    return pl.pallas_call(
        _a2a_body,
        out_shape=jax.ShapeDtypeStruct(x_sorted.shape, x_sorted.dtype),
        in_specs=[
            pl.BlockSpec(memory_space=pltpu.VMEM),
            pl.BlockSpec(memory_space=pltpu.SMEM),
        ],
        out_specs=pl.BlockSpec(memory_space=pltpu.VMEM),
        scratch_shapes=[
            pltpu.SemaphoreType.DMA,
            pltpu.SemaphoreType.DMA,
            pltpu.SemaphoreType.DMA,
            pltpu.SemaphoreType.DMA,
            pltpu.SemaphoreType.DMA,
            pltpu.SemaphoreType.DMA,
        ],
        compiler_params=_CompilerParams(collective_id=1),
    )(x_sorted, cm)


def kernel(x, dest):
    me = lax.axis_index("i")
    dest = dest.astype(jnp.int32)

    m, n = x.shape
    iota = jnp.arange(m, dtype=jnp.int32)
    perm = jnp.sort(dest * jnp.int32(m) + iota) % jnp.int32(m)
    x_sorted = x.astype(jnp.bfloat16)[perm].reshape(m, 8, n // 8)

    counts = jnp.sum(
        dest[:, None] == jnp.arange(N_DEV, dtype=jnp.int32)[None, :],
        axis=0,
        dtype=jnp.int32,
    )
    counts_row = jnp.zeros((1, 8, 128), jnp.int32).at[0, 0, :N_DEV].set(counts)
    cm = _gather_counts(counts_row)[:, 0, :N_DEV]

    out = _a2a(x_sorted, cm)
    return out.reshape(m, n)


# device time: 94528 ns/iter; 1.2085x vs baseline; 1.0074x over previous
import jax
import jax.numpy as jnp
from jax import lax
from jax.experimental import pallas as pl
from jax.experimental.pallas import tpu as pltpu

N_DEV = 4
CHUNK = 32

_CompilerParams = getattr(pltpu, "CompilerParams", None) or getattr(
    pltpu, "TPUCompilerParams"
)


def _neighbor_barrier(me):
    barrier = pltpu.get_barrier_semaphore()
    for d in range(1, N_DEV):
        pl.semaphore_signal(
            barrier,
            inc=1,
            device_id=((me + d) % N_DEV,),
            device_id_type=pl.DeviceIdType.MESH,
        )
    pl.semaphore_wait(barrier, N_DEV - 1)


def _counts_body(cnt_ref, out_ref, local_sem, send_sems, recv_sems):
    me = lax.axis_index("i")
    _neighbor_barrier(me)

    own = pltpu.make_async_copy(cnt_ref, out_ref.at[pl.ds(me, 1)], local_sem)
    own.start()

    rdmas = []
    for d in range(1, N_DEV):
        p = (me + d) % N_DEV
        r = pltpu.make_async_remote_copy(
            src_ref=cnt_ref,
            dst_ref=out_ref.at[pl.ds(me, 1)],
            send_sem=send_sems.at[d - 1],
            recv_sem=recv_sems.at[d - 1],
            device_id=(p,),
            device_id_type=pl.DeviceIdType.MESH,
        )
        r.start()
        rdmas.append(r)

    own.wait()
    for r in rdmas:
        r.wait()


def _gather_counts(counts_row):
    return pl.pallas_call(
        _counts_body,
        out_shape=jax.ShapeDtypeStruct((N_DEV, 8, 128), jnp.int32),
        in_specs=[pl.BlockSpec(memory_space=pltpu.VMEM)],
        out_specs=pl.BlockSpec(memory_space=pltpu.VMEM),
        scratch_shapes=[
            pltpu.SemaphoreType.DMA,
            pltpu.SemaphoreType.DMA((N_DEV - 1,)),
            pltpu.SemaphoreType.DMA((N_DEV - 1,)),
        ],
        compiler_params=_CompilerParams(collective_id=0),
    )(counts_row)


def _a2a_body(
    x_ref,
    cm_ref,
    out_ref,
    send_big,
    send_small,
    recv_big,
    recv_small,
    local_big,
    local_small,
):
    me = lax.axis_index("i")
    _neighbor_barrier(me)

    C = CHUNK
    zero = jnp.int32(0)

    def lstart(p):
        acc = zero
        for d in range(N_DEV):
            acc = acc + jnp.where(jnp.int32(d) < p, cm_ref[me, d], zero)
        return acc

    def rbase(p):
        acc = zero
        for s in range(N_DEV):
            acc = acc + jnp.where(jnp.int32(s) < me, cm_ref[s, p], zero)
        return acc

    nb_send = zero
    ns_send = zero
    for d in range(1, N_DEV):
        p = (me + d) % N_DEV
        c = cm_ref[me, p]
        s0 = lstart(p)
        t0 = rbase(p)
        nb = c // C
        rem = c - nb * C

        def send_chunk(j, _, p=p, s0=s0, t0=t0):
            pltpu.make_async_remote_copy(
                src_ref=x_ref.at[pl.ds(s0 + j * C, C)],
                dst_ref=out_ref.at[pl.ds(t0 + j * C, C)],
                send_sem=send_big,
                recv_sem=recv_big,
                device_id=(p,),
                device_id_type=pl.DeviceIdType.MESH,
            ).start()
            return 0

        lax.fori_loop(0, nb, send_chunk, 0)

        def send_row(k, _, p=p, s0=s0, t0=t0, base=nb * C):
            pltpu.make_async_remote_copy(
                src_ref=x_ref.at[pl.ds(s0 + base + k, 1)],
                dst_ref=out_ref.at[pl.ds(t0 + base + k, 1)],
                send_sem=send_small,
                recv_sem=recv_small,
                device_id=(p,),
                device_id_type=pl.DeviceIdType.MESH,
            ).start()
            return 0

        lax.fori_loop(0, rem, send_row, 0)
        nb_send = nb_send + nb
        ns_send = ns_send + rem

    c_self = cm_ref[me, me]
    s0_self = lstart(me)
    t0_self = rbase(me)
    nb_self = c_self // C
    rem_self = c_self - nb_self * C
    base_self = nb_self * C

    def loc_chunk(j, _):
        pltpu.make_async_copy(
            x_ref.at[pl.ds(s0_self + j * C, C)],
            out_ref.at[pl.ds(t0_self + j * C, C)],
            local_big,
        ).start()
        return 0

    lax.fori_loop(0, nb_self, loc_chunk, 0)

    def loc_row(k, _):
        pltpu.make_async_copy(
            x_ref.at[pl.ds(s0_self + base_self + k, 1)],
            out_ref.at[pl.ds(t0_self + base_self + k, 1)],
            local_small,
        ).start()
        return 0

    lax.fori_loop(0, rem_self, loc_row, 0)

    big_dummy = pltpu.make_async_remote_copy(
        src_ref=x_ref.at[pl.ds(0, C)],
        dst_ref=out_ref.at[pl.ds(0, C)],
        send_sem=send_big,
        recv_sem=recv_big,
        device_id=(me,),
        device_id_type=pl.DeviceIdType.MESH,
    )
    small_dummy = pltpu.make_async_remote_copy(
        src_ref=x_ref.at[pl.ds(0, 1)],
        dst_ref=out_ref.at[pl.ds(0, 1)],
        send_sem=send_small,
        recv_sem=recv_small,
        device_id=(me,),
        device_id_type=pl.DeviceIdType.MESH,
    )

    def wait_send_big(i, _):
        big_dummy.wait_send()
        return 0

    def wait_send_small(i, _):
        small_dummy.wait_send()
        return 0

    def wait_recv_big(i, _):
        big_dummy.wait_recv()
        return 0

    def wait_recv_small(i, _):
        small_dummy.wait_recv()
        return 0

    nbig_in = zero
    nsmall_in = zero
    for s in range(N_DEV):
        c_in = cm_ref[s, me]
        is_remote = jnp.int32(s) != me
        nbig_in = nbig_in + jnp.where(is_remote, c_in // C, zero)
        nsmall_in = nsmall_in + jnp.where(is_remote, c_in % C, zero)

    lax.fori_loop(0, nb_send, wait_send_big, 0)
    lax.fori_loop(0, ns_send, wait_send_small, 0)
    lax.fori_loop(0, nbig_in, wait_recv_big, 0)
    lax.fori_loop(0, nsmall_in, wait_recv_small, 0)

    def wait_loc_big(i, _):
        pltpu.make_async_copy(
            x_ref.at[pl.ds(0, C)], out_ref.at[pl.ds(0, C)], local_big
        ).wait()
        return 0

    def wait_loc_small(i, _):
        pltpu.make_async_copy(
            x_ref.at[pl.ds(0, 1)], out_ref.at[pl.ds(0, 1)], local_small
        ).wait()
        return 0

    lax.fori_loop(0, nb_self, wait_loc_big, 0)
    lax.fori_loop(0, rem_self, wait_loc_small, 0)


def _a2a(x_sorted, cm):
    return pl.pallas_call(
        _a2a_body,
        out_shape=jax.ShapeDtypeStruct(x_sorted.shape, x_sorted.dtype),
        in_specs=[
            pl.BlockSpec(memory_space=pltpu.VMEM),
            pl.BlockSpec(memory_space=pltpu.SMEM),
        ],
        out_specs=pl.BlockSpec(memory_space=pltpu.VMEM),
        scratch_shapes=[
            pltpu.SemaphoreType.DMA,
            pltpu.SemaphoreType.DMA,
            pltpu.SemaphoreType.DMA,
            pltpu.SemaphoreType.DMA,
            pltpu.SemaphoreType.DMA,
            pltpu.SemaphoreType.DMA,
        ],
        compiler_params=_CompilerParams(collective_id=1),
    )(x_sorted, cm)


def kernel(x, dest):
    me = lax.axis_index("i")
    dest = dest.astype(jnp.int32)

    m, n = x.shape
    iota = jnp.arange(m, dtype=jnp.int32)
    perm = jnp.sort(dest * jnp.int32(m) + iota) % jnp.int32(m)
    x_sorted = x.astype(jnp.bfloat16)[perm].reshape(m, 8, n // 8)

    counts = jnp.sum(
        dest[:, None] == jnp.arange(N_DEV, dtype=jnp.int32)[None, :],
        axis=0,
        dtype=jnp.int32,
    )
    counts_row = jnp.zeros((1, 8, 128), jnp.int32).at[0, 0, :N_DEV].set(counts)
    cm = _gather_counts(counts_row)[:, 0, :N_DEV]

    out = _a2a(x_sorted, cm)
    return out.reshape(m, n)
